# baseline (speedup 1.0000x reference)
import functools

import jax
import jax.numpy as jnp
from jax import lax
from jax.experimental import pallas as pl
from jax.experimental.pallas import tpu as pltpu
from jax.experimental.pallas import tpu_sc as plsc

_M = 4


def _rot_reg_kernel(x_ref, rt_ref, r_ref, xr_ref, reg_ref):
    m = pl.program_id(0)
    i = pl.program_id(1)
    xr_ref[0] = jnp.dot(x_ref[...], rt_ref[0], preferred_element_type=jnp.float32)

    @pl.when((m == 0) & (i == 0))
    def _():
        r = r_ref[...]
        d = r.shape[0]
        rrt = lax.dot_general(r, r, (((1,), (1,)), ((), ())),
                              preferred_element_type=jnp.float32)
        eye = jnp.eye(d, dtype=jnp.float32)
        reg_ref[...] = (jnp.sum((rrt - eye) ** 2) / (d * d)).reshape(1, 1)


def _vq_kernel(xs_ref, cbt_ref, cb_ref, codes_ref, soft_ref):
    xs = xs_ref[0]
    cbt = cbt_ref[0]
    cb = cb_ref[0]
    cc = jnp.sum(cbt * cbt, axis=0, keepdims=True)
    logits = 2.0 * jnp.dot(xs, cbt, preferred_element_type=jnp.float32) - cc
    mx = jnp.max(logits, axis=-1, keepdims=True)
    e = jnp.exp(logits - mx)
    l = jnp.sum(e, axis=-1, keepdims=True)
    soft_ref[0] = jnp.dot(e.astype(jnp.bfloat16), cb.astype(jnp.bfloat16),
                          preferred_element_type=jnp.float32) / l
    k = logits.shape[-1]
    idx = lax.broadcasted_iota(jnp.int32, logits.shape, 1)
    amax = jnp.min(jnp.where(logits >= mx, idx, k), axis=-1)
    codes_ref[0, 0] = amax


def _loss_kernel(xs_ref, soft_ref, hard_ref, reg_ref, out_ref, acc_ref):
    m = pl.program_id(0)
    n = pl.program_id(1)
    nm = pl.num_programs(0)
    nn = pl.num_programs(1)
    split = xs_ref[0]
    soft = soft_ref[0]
    hard = hard_ref[0][:, : split.shape[1]]

    @pl.when((m == 0) & (n == 0))
    def _():
        acc_ref[0] = 0.0
        acc_ref[1] = 0.0
        acc_ref[2] = 0.0

    acc_ref[0] += jnp.sum((split - soft) ** 2)
    acc_ref[1] += jnp.sum((split - hard) ** 2)
    acc_ref[2] += jnp.sum((soft - hard) ** 2)

    @pl.when((m == nm - 1) & (n == nn - 1))
    def _():
        cnt = nn * split.shape[0] * split.shape[1]
        loss = (0.1 * acc_ref[0] + acc_ref[1] + 0.1 * acc_ref[2]) / cnt
        out_ref[...] = loss + 0.01 * reg_ref[...]


def _make_sc_gather(tot, s, n_rows, k_rows):
    info = plsc.get_sparse_core_info()
    nc, ns = info.num_cores, info.num_subcores
    nw = nc * ns
    items_pw = tot // nw
    ch = min(128, items_pw)
    nch = items_pw // ch
    mesh = plsc.VectorSubcoreMesh(core_axis_name="c", subcore_axis_name="s")

    @functools.partial(
        pl.kernel, mesh=mesh,
        out_type=jax.ShapeDtypeStruct((tot, s), jnp.float32),
        scratch_types=[
            pltpu.VMEM((ch,), jnp.int32),
            pltpu.VMEM((ch, s), jnp.float32),
            pltpu.SemaphoreType.DMA,
        ],
    )
    def k(codes_hbm, table_hbm, out_hbm, idx_v, rows_v, sem):
        wid = lax.axis_index("s") * nc + lax.axis_index("c")
        base = wid * items_pw
        moff = (base // n_rows) * k_rows
        for c in range(nch):
            off = base + c * ch
            pltpu.sync_copy(codes_hbm.at[pl.ds(off, ch)], idx_v)
            for v in range(ch // 16):
                sl = pl.ds(v * 16, 16)
                idx_v[sl] = idx_v[sl] + moff
            pltpu.async_copy(table_hbm.at[idx_v], rows_v, sem).wait()
            pltpu.sync_copy(rows_v, out_hbm.at[pl.ds(off, ch)])

    return k


def kernel(x, codebook0, codebook1, codebook2, codebook3, rotateMatrix):
    n, d = x.shape
    cbs = jnp.stack([codebook0, codebook1, codebook2, codebook3])
    m_, k, s = cbs.shape
    cbt = cbs.transpose(0, 2, 1)

    rt = rotateMatrix.reshape(d, m_, s).transpose(1, 0, 2)
    bn1 = 512
    xrs, reg = pl.pallas_call(
        _rot_reg_kernel,
        grid=(m_, n // bn1),
        in_specs=[
            pl.BlockSpec((bn1, d), lambda m, i: (i, 0)),
            pl.BlockSpec((1, d, s), lambda m, i: (m, 0, 0)),
            pl.BlockSpec((d, d), lambda m, i: (0, 0)),
        ],
        out_specs=[
            pl.BlockSpec((1, bn1, s), lambda m, i: (m, i, 0)),
            pl.BlockSpec((1, 1), lambda m, i: (0, 0)),
        ],
        out_shape=[
            jax.ShapeDtypeStruct((m_, n, s), jnp.float32),
            jax.ShapeDtypeStruct((1, 1), jnp.float32),
        ],
    )(x, rt, rotateMatrix)

    bn = 512
    codes, soft = pl.pallas_call(
        _vq_kernel,
        grid=(m_, n // bn),
        in_specs=[
            pl.BlockSpec((1, bn, s), lambda m, i: (m, i, 0)),
            pl.BlockSpec((1, s, k), lambda m, i: (m, 0, 0)),
            pl.BlockSpec((1, k, s), lambda m, i: (m, 0, 0)),
        ],
        out_specs=[
            pl.BlockSpec((1, 1, bn), lambda m, i: (m, 0, i)),
            pl.BlockSpec((1, bn, s), lambda m, i: (m, i, 0)),
        ],
        out_shape=[
            jax.ShapeDtypeStruct((m_, 1, n), jnp.int32),
            jax.ShapeDtypeStruct((m_, n, s), jnp.float32),
        ],
    )(xrs, cbt, cbs)

    sp = 128
    codes_flat = codes.reshape(m_ * n)
    table = jnp.pad(cbs.reshape(m_ * k, s), ((0, 0), (0, sp - s)))
    hard = _make_sc_gather(m_ * n, sp, n, k)(codes_flat, table)
    hard = hard.reshape(m_, n, sp)

    bn3 = 512
    loss = pl.pallas_call(
        _loss_kernel,
        grid=(m_, n // bn3),
        in_specs=[
            pl.BlockSpec((1, bn3, s), lambda m, i: (m, i, 0)),
            pl.BlockSpec((1, bn3, s), lambda m, i: (m, i, 0)),
            pl.BlockSpec((1, bn3, sp), lambda m, i: (m, i, 0)),
            pl.BlockSpec((1, 1), lambda m, i: (0, 0)),
        ],
        out_specs=pl.BlockSpec((1, 1), lambda m, i: (0, 0)),
        out_shape=jax.ShapeDtypeStruct((1, 1), jnp.float32),
        scratch_shapes=[pltpu.SMEM((3,), jnp.float32)],
    )(xrs, soft, hard, reg)

    hard_codes = codes.reshape(m_, n).T
    return (hard_codes, loss[0, 0])


# speedup vs baseline: 1.1919x; 1.1919x over previous
import functools

import jax
import jax.numpy as jnp
from jax import lax
from jax.experimental import pallas as pl
from jax.experimental.pallas import tpu as pltpu
from jax.experimental.pallas import tpu_sc as plsc

_M = 4


def _rot_reg_kernel(x_ref, rt_ref, r_ref, xr_ref, reg_ref):
    m = pl.program_id(0)
    i = pl.program_id(1)
    xr_ref[0] = jnp.dot(x_ref[...], rt_ref[0], preferred_element_type=jnp.float32)

    @pl.when((m == 0) & (i == 0))
    def _():
        r = r_ref[...]
        d = r.shape[0]
        rrt = lax.dot_general(r, r, (((1,), (1,)), ((), ())),
                              preferred_element_type=jnp.float32)
        eye = jnp.eye(d, dtype=jnp.float32)
        reg_ref[...] = (jnp.sum((rrt - eye) ** 2) / (d * d)).reshape(1, 1)


def _vq_kernel(xs_ref, cbt_ref, cb_ref, codes_ref, soft_ref):
    xs = xs_ref[0]
    cbt = cbt_ref[0]
    cb = cb_ref[0]
    cc = jnp.sum(cbt * cbt, axis=0, keepdims=True)
    logits = 2.0 * jnp.dot(xs, cbt, preferred_element_type=jnp.float32) - cc
    mx = jnp.max(logits, axis=-1, keepdims=True)
    e = jnp.exp(logits - mx)
    l = jnp.sum(e, axis=-1, keepdims=True)
    soft_ref[0] = jnp.dot(e.astype(jnp.bfloat16), cb.astype(jnp.bfloat16),
                          preferred_element_type=jnp.float32) / l
    k = logits.shape[-1]
    idx = lax.broadcasted_iota(jnp.int32, logits.shape, 1)
    amax = jnp.min(jnp.where(logits >= mx, idx, k), axis=-1)
    codes_ref[0, 0] = amax


def _loss_kernel(xs_ref, soft_ref, hard_ref, reg_ref, out_ref, acc_ref):
    m = pl.program_id(0)
    n = pl.program_id(1)
    nm = pl.num_programs(0)
    nn = pl.num_programs(1)
    split = xs_ref[0]
    soft = soft_ref[0]
    hard = hard_ref[0][:, : split.shape[1]]

    @pl.when((m == 0) & (n == 0))
    def _():
        acc_ref[0] = 0.0
        acc_ref[1] = 0.0
        acc_ref[2] = 0.0

    acc_ref[0] += jnp.sum((split - soft) ** 2)
    acc_ref[1] += jnp.sum((split - hard) ** 2)
    acc_ref[2] += jnp.sum((soft - hard) ** 2)

    @pl.when((m == nm - 1) & (n == nn - 1))
    def _():
        cnt = nn * split.shape[0] * split.shape[1]
        loss = (0.1 * acc_ref[0] + acc_ref[1] + 0.1 * acc_ref[2]) / cnt
        out_ref[...] = loss + 0.01 * reg_ref[...]


def _make_sc_gather(tot, s, n_rows, k_rows):
    info = plsc.get_sparse_core_info()
    nc, ns = info.num_cores, info.num_subcores
    nw = nc * ns
    items_pw = tot // nw
    ch = min(128, items_pw)
    nch = items_pw // ch
    mesh = plsc.VectorSubcoreMesh(core_axis_name="c", subcore_axis_name="s")

    @functools.partial(
        pl.kernel, mesh=mesh,
        out_type=jax.ShapeDtypeStruct((tot, s), jnp.float32),
        scratch_types=[
            pltpu.VMEM((ch,), jnp.int32),
            pltpu.VMEM((ch, s), jnp.float32),
            pltpu.SemaphoreType.DMA,
        ],
    )
    def k(codes_hbm, table_hbm, out_hbm, idx_v, rows_v, sem):
        wid = lax.axis_index("s") * nc + lax.axis_index("c")
        base = wid * items_pw
        moff = (base // n_rows) * k_rows
        for c in range(nch):
            off = base + c * ch
            pltpu.sync_copy(codes_hbm.at[pl.ds(off, ch)], idx_v)
            for v in range(ch // 16):
                sl = pl.ds(v * 16, 16)
                idx_v[sl] = idx_v[sl] + moff
            pltpu.async_copy(table_hbm.at[idx_v], rows_v, sem).wait()
            pltpu.sync_copy(rows_v, out_hbm.at[pl.ds(off, ch)])

    return k


def kernel(x, codebook0, codebook1, codebook2, codebook3, rotateMatrix):
    n, d = x.shape
    cbs = jnp.stack([codebook0, codebook1, codebook2, codebook3])
    m_, k, s = cbs.shape
    cbt = cbs.transpose(0, 2, 1)

    rt = rotateMatrix.reshape(d, m_, s).transpose(1, 0, 2)
    bn1 = 512
    xrs, reg = pl.pallas_call(
        _rot_reg_kernel,
        grid=(m_, n // bn1),
        in_specs=[
            pl.BlockSpec((bn1, d), lambda m, i: (i, 0)),
            pl.BlockSpec((1, d, s), lambda m, i: (m, 0, 0)),
            pl.BlockSpec((d, d), lambda m, i: (0, 0)),
        ],
        out_specs=[
            pl.BlockSpec((1, bn1, s), lambda m, i: (m, i, 0)),
            pl.BlockSpec((1, 1), lambda m, i: (0, 0)),
        ],
        out_shape=[
            jax.ShapeDtypeStruct((m_, n, s), jnp.float32),
            jax.ShapeDtypeStruct((1, 1), jnp.float32),
        ],
    )(x, rt, rotateMatrix)

    bn = 128
    codes, soft = pl.pallas_call(
        _vq_kernel,
        grid=(m_, n // bn),
        in_specs=[
            pl.BlockSpec((1, bn, s), lambda m, i: (m, i, 0)),
            pl.BlockSpec((1, s, k), lambda m, i: (m, 0, 0)),
            pl.BlockSpec((1, k, s), lambda m, i: (m, 0, 0)),
        ],
        out_specs=[
            pl.BlockSpec((1, 1, bn), lambda m, i: (m, 0, i)),
            pl.BlockSpec((1, bn, s), lambda m, i: (m, i, 0)),
        ],
        out_shape=[
            jax.ShapeDtypeStruct((m_, 1, n), jnp.int32),
            jax.ShapeDtypeStruct((m_, n, s), jnp.float32),
        ],
    )(xrs, cbt, cbs)

    sp = 128
    codes_flat = codes.reshape(m_ * n)
    table = jnp.pad(cbs.reshape(m_ * k, s), ((0, 0), (0, sp - s)))
    hard = _make_sc_gather(m_ * n, sp, n, k)(codes_flat, table)
    hard = hard.reshape(m_, n, sp)

    bn3 = 512
    loss = pl.pallas_call(
        _loss_kernel,
        grid=(m_, n // bn3),
        in_specs=[
            pl.BlockSpec((1, bn3, s), lambda m, i: (m, i, 0)),
            pl.BlockSpec((1, bn3, s), lambda m, i: (m, i, 0)),
            pl.BlockSpec((1, bn3, sp), lambda m, i: (m, i, 0)),
            pl.BlockSpec((1, 1), lambda m, i: (0, 0)),
        ],
        out_specs=pl.BlockSpec((1, 1), lambda m, i: (0, 0)),
        out_shape=jax.ShapeDtypeStruct((1, 1), jnp.float32),
        scratch_shapes=[pltpu.SMEM((3,), jnp.float32)],
    )(xrs, soft, hard, reg)

    hard_codes = codes.reshape(m_, n).T
    return (hard_codes, loss[0, 0])


# speedup vs baseline: 1.2744x; 1.0692x over previous
import functools

import jax
import jax.numpy as jnp
from jax import lax
from jax.experimental import pallas as pl
from jax.experimental.pallas import tpu as pltpu
from jax.experimental.pallas import tpu_sc as plsc

_M = 4


def _rot_reg_kernel(x_ref, rt_ref, r_ref, xr_ref, reg_ref):
    m = pl.program_id(0)
    i = pl.program_id(1)
    xr_ref[0] = jnp.dot(x_ref[...], rt_ref[0], preferred_element_type=jnp.float32)

    @pl.when((m == 0) & (i == 0))
    def _():
        r = r_ref[...]
        d = r.shape[0]
        rrt = lax.dot_general(r, r, (((1,), (1,)), ((), ())),
                              preferred_element_type=jnp.float32)
        eye = jnp.eye(d, dtype=jnp.float32)
        reg_ref[...] = (jnp.sum((rrt - eye) ** 2) / (d * d)).reshape(1, 1)


def _vq_kernel(xs_ref, cbt_ref, cba_ref, codes_ref, soft_ref):
    xs = 2.0 * xs_ref[0]
    cbt = cbt_ref[0]
    cba = cba_ref[0]
    s = xs.shape[-1]
    cc = jnp.sum(cbt * cbt, axis=0, keepdims=True)
    logits = jnp.dot(xs, cbt, preferred_element_type=jnp.float32) - cc
    mx = jnp.max(logits, axis=-1, keepdims=True)
    e = jnp.exp(logits - mx)
    acc = jnp.dot(e.astype(jnp.bfloat16), cba, preferred_element_type=jnp.float32)
    soft_ref[0] = acc[:, :s] / acc[:, s:s + 1]
    k = logits.shape[-1]
    idx = lax.broadcasted_iota(jnp.int32, logits.shape, 1)
    amax = jnp.min(jnp.where(logits >= mx, idx, k), axis=-1)
    codes_ref[0, 0] = amax


def _loss_kernel(xs_ref, soft_ref, hard_ref, reg_ref, out_ref, acc_ref):
    m = pl.program_id(0)
    n = pl.program_id(1)
    nm = pl.num_programs(0)
    nn = pl.num_programs(1)
    split = xs_ref[0]
    soft = soft_ref[0]
    hard = hard_ref[0][:, : split.shape[1]]

    @pl.when((m == 0) & (n == 0))
    def _():
        acc_ref[0] = 0.0
        acc_ref[1] = 0.0
        acc_ref[2] = 0.0

    acc_ref[0] += jnp.sum((split - soft) ** 2)
    acc_ref[1] += jnp.sum((split - hard) ** 2)
    acc_ref[2] += jnp.sum((soft - hard) ** 2)

    @pl.when((m == nm - 1) & (n == nn - 1))
    def _():
        cnt = nn * split.shape[0] * split.shape[1]
        loss = (0.1 * acc_ref[0] + acc_ref[1] + 0.1 * acc_ref[2]) / cnt
        out_ref[...] = loss + 0.01 * reg_ref[...]


def _make_sc_gather(tot, s, n_rows, k_rows):
    info = plsc.get_sparse_core_info()
    nc, ns = info.num_cores, info.num_subcores
    nw = nc * ns
    items_pw = tot // nw
    ch = min(128, items_pw)
    nch = items_pw // ch
    mesh = plsc.VectorSubcoreMesh(core_axis_name="c", subcore_axis_name="s")

    @functools.partial(
        pl.kernel, mesh=mesh,
        out_type=jax.ShapeDtypeStruct((tot, s), jnp.float32),
        scratch_types=[
            pltpu.VMEM((ch,), jnp.int32),
            pltpu.VMEM((ch, s), jnp.float32),
            pltpu.SemaphoreType.DMA,
        ],
    )
    def k(codes_hbm, table_hbm, out_hbm, idx_v, rows_v, sem):
        wid = lax.axis_index("s") * nc + lax.axis_index("c")
        base = wid * items_pw
        moff = (base // n_rows) * k_rows
        for c in range(nch):
            off = base + c * ch
            pltpu.sync_copy(codes_hbm.at[pl.ds(off, ch)], idx_v)
            for v in range(ch // 16):
                sl = pl.ds(v * 16, 16)
                idx_v[sl] = idx_v[sl] + moff
            pltpu.async_copy(table_hbm.at[idx_v], rows_v, sem).wait()
            pltpu.sync_copy(rows_v, out_hbm.at[pl.ds(off, ch)])

    return k


def kernel(x, codebook0, codebook1, codebook2, codebook3, rotateMatrix):
    n, d = x.shape
    cbs = jnp.stack([codebook0, codebook1, codebook2, codebook3])
    m_, k, s = cbs.shape
    cbt = cbs.transpose(0, 2, 1)

    rt = rotateMatrix.reshape(d, m_, s).transpose(1, 0, 2)
    bn1 = 512
    xrs, reg = pl.pallas_call(
        _rot_reg_kernel,
        grid=(m_, n // bn1),
        in_specs=[
            pl.BlockSpec((bn1, d), lambda m, i: (i, 0)),
            pl.BlockSpec((1, d, s), lambda m, i: (m, 0, 0)),
            pl.BlockSpec((d, d), lambda m, i: (0, 0)),
        ],
        out_specs=[
            pl.BlockSpec((1, bn1, s), lambda m, i: (m, i, 0)),
            pl.BlockSpec((1, 1), lambda m, i: (0, 0)),
        ],
        out_shape=[
            jax.ShapeDtypeStruct((m_, n, s), jnp.float32),
            jax.ShapeDtypeStruct((1, 1), jnp.float32),
        ],
    )(x, rt, rotateMatrix)

    spad = 128
    cba = jnp.zeros((m_, k, spad), jnp.bfloat16)
    cba = cba.at[:, :, :s].set(cbs.astype(jnp.bfloat16))
    cba = cba.at[:, :, s].set(1.0)
    bn = 256
    codes, soft = pl.pallas_call(
        _vq_kernel,
        grid=(m_, n // bn),
        in_specs=[
            pl.BlockSpec((1, bn, s), lambda m, i: (m, i, 0)),
            pl.BlockSpec((1, s, k), lambda m, i: (m, 0, 0)),
            pl.BlockSpec((1, k, spad), lambda m, i: (m, 0, 0)),
        ],
        out_specs=[
            pl.BlockSpec((1, 1, bn), lambda m, i: (m, 0, i)),
            pl.BlockSpec((1, bn, s), lambda m, i: (m, i, 0)),
        ],
        out_shape=[
            jax.ShapeDtypeStruct((m_, 1, n), jnp.int32),
            jax.ShapeDtypeStruct((m_, n, s), jnp.float32),
        ],
    )(xrs, cbt, cba)

    sp = 128
    codes_flat = codes.reshape(m_ * n)
    table = jnp.pad(cbs.reshape(m_ * k, s), ((0, 0), (0, sp - s)))
    hard = _make_sc_gather(m_ * n, sp, n, k)(codes_flat, table)
    hard = hard.reshape(m_, n, sp)

    bn3 = 512
    loss = pl.pallas_call(
        _loss_kernel,
        grid=(m_, n // bn3),
        in_specs=[
            pl.BlockSpec((1, bn3, s), lambda m, i: (m, i, 0)),
            pl.BlockSpec((1, bn3, s), lambda m, i: (m, i, 0)),
            pl.BlockSpec((1, bn3, sp), lambda m, i: (m, i, 0)),
            pl.BlockSpec((1, 1), lambda m, i: (0, 0)),
        ],
        out_specs=pl.BlockSpec((1, 1), lambda m, i: (0, 0)),
        out_shape=jax.ShapeDtypeStruct((1, 1), jnp.float32),
        scratch_shapes=[pltpu.SMEM((3,), jnp.float32)],
    )(xrs, soft, hard, reg)

    hard_codes = codes.reshape(m_, n).T
    return (hard_codes, loss[0, 0])


# speedup vs baseline: 1.4760x; 1.1582x over previous
import functools

import jax
import jax.numpy as jnp
from jax import lax
from jax.experimental import pallas as pl
from jax.experimental.pallas import tpu as pltpu
from jax.experimental.pallas import tpu_sc as plsc

_M = 4


def _rot_reg_kernel(x_ref, rt_ref, r_ref, xr_ref, reg_ref):
    i = pl.program_id(0)
    m = pl.program_id(1)
    xr_ref[0] = jnp.dot(x_ref[...], rt_ref[0], preferred_element_type=jnp.float32)

    @pl.when((m == 0) & (i == 0))
    def _():
        r = r_ref[...]
        d = r.shape[0]
        rrt = lax.dot_general(r, r, (((1,), (1,)), ((), ())),
                              preferred_element_type=jnp.float32)
        eye = jnp.eye(d, dtype=jnp.float32)
        reg_ref[...] = (jnp.sum((rrt - eye) ** 2) / (d * d)).reshape(1, 1)


def _vq_kernel(xs_ref, cbt_ref, cba_ref, codes_ref, soft_ref):
    xs = 2.0 * xs_ref[0]
    cbt = cbt_ref[0]
    cba = cba_ref[0]
    s = xs.shape[-1]
    cc = jnp.sum(cbt * cbt, axis=0, keepdims=True)
    logits = jnp.dot(xs, cbt, preferred_element_type=jnp.float32) - cc
    mx = jnp.max(logits, axis=-1, keepdims=True)
    e = jnp.exp(logits - mx)
    acc = jnp.dot(e.astype(jnp.bfloat16), cba, preferred_element_type=jnp.float32)
    soft_ref[0] = acc[:, :s] / acc[:, s:s + 1]
    k = logits.shape[-1]
    idx = lax.broadcasted_iota(jnp.int32, logits.shape, 1)
    amax = jnp.min(jnp.where(logits >= mx, idx, k), axis=-1)
    codes_ref[0, 0] = amax


def _loss_kernel(xs_ref, soft_ref, hard_ref, reg_ref, out_ref, acc_ref):
    m = pl.program_id(0)
    n = pl.program_id(1)
    nm = pl.num_programs(0)
    nn = pl.num_programs(1)
    split = xs_ref[0]
    soft = soft_ref[0]
    hard = hard_ref[0][:, : split.shape[1]]

    @pl.when((m == 0) & (n == 0))
    def _():
        acc_ref[0] = 0.0
        acc_ref[1] = 0.0
        acc_ref[2] = 0.0

    acc_ref[0] += jnp.sum((split - soft) ** 2)
    acc_ref[1] += jnp.sum((split - hard) ** 2)
    acc_ref[2] += jnp.sum((soft - hard) ** 2)

    @pl.when((m == nm - 1) & (n == nn - 1))
    def _():
        cnt = nn * split.shape[0] * split.shape[1]
        loss = (0.1 * acc_ref[0] + acc_ref[1] + 0.1 * acc_ref[2]) / cnt
        out_ref[...] = loss + 0.01 * reg_ref[...]


def _make_sc_gather(tot, s, n_rows, k_rows):
    info = plsc.get_sparse_core_info()
    nc, ns = info.num_cores, info.num_subcores
    nw = nc * ns
    items_pw = tot // nw
    ch = min(128, items_pw)
    nch = items_pw // ch
    mesh = plsc.VectorSubcoreMesh(core_axis_name="c", subcore_axis_name="s")

    @functools.partial(
        pl.kernel, mesh=mesh,
        out_type=jax.ShapeDtypeStruct((tot, s), jnp.float32),
        scratch_types=[
            pltpu.VMEM((ch,), jnp.int32),
            pltpu.VMEM((ch, s), jnp.float32),
            pltpu.SemaphoreType.DMA,
        ],
    )
    def k(codes_hbm, table_hbm, out_hbm, idx_v, rows_v, sem):
        wid = lax.axis_index("s") * nc + lax.axis_index("c")
        base = wid * items_pw
        moff = (base // n_rows) * k_rows
        for c in range(nch):
            off = base + c * ch
            pltpu.sync_copy(codes_hbm.at[pl.ds(off, ch)], idx_v)
            for v in range(ch // 16):
                sl = pl.ds(v * 16, 16)
                idx_v[sl] = idx_v[sl] + moff
            pltpu.async_copy(table_hbm.at[idx_v], rows_v, sem).wait()
            pltpu.sync_copy(rows_v, out_hbm.at[pl.ds(off, ch)])

    return k


def kernel(x, codebook0, codebook1, codebook2, codebook3, rotateMatrix):
    n, d = x.shape
    cbs = jnp.stack([codebook0, codebook1, codebook2, codebook3])
    m_, k, s = cbs.shape
    cbt = cbs.transpose(0, 2, 1)

    rt = rotateMatrix.reshape(d, m_, s).transpose(1, 0, 2)
    bn1 = min(1024, n)
    xrs, reg = pl.pallas_call(
        _rot_reg_kernel,
        grid=(n // bn1, m_),
        in_specs=[
            pl.BlockSpec((bn1, d), lambda i, m: (i, 0)),
            pl.BlockSpec((1, d, s), lambda i, m: (m, 0, 0)),
            pl.BlockSpec((d, d), lambda i, m: (0, 0)),
        ],
        out_specs=[
            pl.BlockSpec((1, bn1, s), lambda i, m: (m, i, 0)),
            pl.BlockSpec((1, 1), lambda i, m: (0, 0)),
        ],
        out_shape=[
            jax.ShapeDtypeStruct((m_, n, s), jnp.float32),
            jax.ShapeDtypeStruct((1, 1), jnp.float32),
        ],
    )(x, rt, rotateMatrix)

    spad = 128
    cba = jnp.concatenate(
        [cbs.astype(jnp.bfloat16),
         jnp.ones((m_, k, 1), jnp.bfloat16),
         jnp.zeros((m_, k, spad - s - 1), jnp.bfloat16)], axis=2)
    bn = min(256, n)
    codes, soft = pl.pallas_call(
        _vq_kernel,
        grid=(m_, n // bn),
        in_specs=[
            pl.BlockSpec((1, bn, s), lambda m, i: (m, i, 0)),
            pl.BlockSpec((1, s, k), lambda m, i: (m, 0, 0)),
            pl.BlockSpec((1, k, spad), lambda m, i: (m, 0, 0)),
        ],
        out_specs=[
            pl.BlockSpec((1, 1, bn), lambda m, i: (m, 0, i)),
            pl.BlockSpec((1, bn, s), lambda m, i: (m, i, 0)),
        ],
        out_shape=[
            jax.ShapeDtypeStruct((m_, 1, n), jnp.int32),
            jax.ShapeDtypeStruct((m_, n, s), jnp.float32),
        ],
    )(xrs, cbt, cba)

    codes_flat = codes.reshape(m_ * n)
    table = jnp.concatenate(
        [cbs.reshape(m_ * k, s), jnp.zeros((m_ * k, spad - s), jnp.float32)], axis=1)
    hard = _make_sc_gather(m_ * n, spad, n, k)(codes_flat, table)
    hard = hard.reshape(m_, n, spad)

    bn3 = min(2048, n)
    loss = pl.pallas_call(
        _loss_kernel,
        grid=(m_, n // bn3),
        in_specs=[
            pl.BlockSpec((1, bn3, s), lambda m, i: (m, i, 0)),
            pl.BlockSpec((1, bn3, s), lambda m, i: (m, i, 0)),
            pl.BlockSpec((1, bn3, spad), lambda m, i: (m, i, 0)),
            pl.BlockSpec((1, 1), lambda m, i: (0, 0)),
        ],
        out_specs=pl.BlockSpec((1, 1), lambda m, i: (0, 0)),
        out_shape=jax.ShapeDtypeStruct((1, 1), jnp.float32),
        scratch_shapes=[pltpu.SMEM((3,), jnp.float32)],
    )(xrs, soft, hard, reg)

    hard_codes = codes.reshape(m_, n).T
    return (hard_codes, loss[0, 0])


# speedup vs baseline: 1.5528x; 1.0521x over previous
import functools

import jax
import jax.numpy as jnp
from jax import lax
from jax.experimental import pallas as pl
from jax.experimental.pallas import tpu as pltpu
from jax.experimental.pallas import tpu_sc as plsc

_M = 4
_LOG2E = 1.4426950408889634


def _rot_reg_kernel(x_ref, rt_ref, r_ref, xr_ref, reg_ref):
    i = pl.program_id(0)
    m = pl.program_id(1)
    xr_ref[0] = jnp.dot(x_ref[...], rt_ref[0], preferred_element_type=jnp.float32)

    @pl.when((m == 0) & (i == 0))
    def _():
        r = r_ref[...]
        d = r.shape[0]
        rrt = lax.dot_general(r, r, (((1,), (1,)), ((), ())),
                              preferred_element_type=jnp.float32)
        eye = jnp.eye(d, dtype=jnp.float32)
        reg_ref[...] = (jnp.sum((rrt - eye) ** 2) / (d * d)).reshape(1, 1)


def _vq_kernel(xs_ref, cbt_ref, bias_ref, cba_ref, codes_ref, soft_ref):
    xs = xs_ref[0]
    cbt = cbt_ref[0]
    cba = cba_ref[0]
    s = xs.shape[-1]
    logits = jnp.dot(xs, cbt, preferred_element_type=jnp.float32) + bias_ref[0]
    mx = jnp.max(logits, axis=-1, keepdims=True)
    e = jnp.exp2(logits - mx)
    acc = jnp.dot(e.astype(jnp.bfloat16), cba, preferred_element_type=jnp.float32)
    soft_ref[0] = acc[:, :s] / acc[:, s:s + 1]
    k = logits.shape[-1]
    idx = lax.broadcasted_iota(jnp.int32, logits.shape, 1)
    amax = jnp.min(jnp.where(logits >= mx, idx, k), axis=-1)
    codes_ref[0, 0] = amax


def _loss_kernel(xs_ref, soft_ref, hard_ref, reg_ref, out_ref, acc_ref):
    m = pl.program_id(0)
    n = pl.program_id(1)
    nm = pl.num_programs(0)
    nn = pl.num_programs(1)
    soft = soft_ref[0]
    s = soft.shape[-1]
    split = xs_ref[0][:, :s] * (0.5 / _LOG2E)
    hard = hard_ref[0][:, :s]

    @pl.when((m == 0) & (n == 0))
    def _():
        acc_ref[0] = 0.0
        acc_ref[1] = 0.0
        acc_ref[2] = 0.0

    acc_ref[0] += jnp.sum((split - soft) ** 2)
    acc_ref[1] += jnp.sum((split - hard) ** 2)
    acc_ref[2] += jnp.sum((soft - hard) ** 2)

    @pl.when((m == nm - 1) & (n == nn - 1))
    def _():
        cnt = nn * split.shape[0] * split.shape[1]
        loss = (0.1 * acc_ref[0] + acc_ref[1] + 0.1 * acc_ref[2]) / cnt
        out_ref[...] = loss + 0.01 * reg_ref[...]


def _make_sc_gather(tot, s, n_rows, k_rows):
    info = plsc.get_sparse_core_info()
    nc, ns = info.num_cores, info.num_subcores
    nw = nc * ns
    items_pw = tot // nw
    ch = min(128, items_pw)
    nch = items_pw // ch
    mesh = plsc.VectorSubcoreMesh(core_axis_name="c", subcore_axis_name="s")

    @functools.partial(
        pl.kernel, mesh=mesh,
        out_type=jax.ShapeDtypeStruct((tot, s), jnp.float32),
        scratch_types=[
            pltpu.VMEM((ch,), jnp.int32),
            pltpu.VMEM((ch, s), jnp.float32),
            pltpu.SemaphoreType.DMA,
        ],
    )
    def k(codes_hbm, table_hbm, out_hbm, idx_v, rows_v, sem):
        wid = lax.axis_index("s") * nc + lax.axis_index("c")
        base = wid * items_pw
        moff = (base // n_rows) * k_rows
        for c in range(nch):
            off = base + c * ch
            pltpu.sync_copy(codes_hbm.at[pl.ds(off, ch)], idx_v)
            for v in range(ch // 16):
                sl = pl.ds(v * 16, 16)
                idx_v[sl] = idx_v[sl] + moff
            pltpu.async_copy(table_hbm.at[idx_v], rows_v, sem).wait()
            pltpu.sync_copy(rows_v, out_hbm.at[pl.ds(off, ch)])

    return k


def kernel(x, codebook0, codebook1, codebook2, codebook3, rotateMatrix):
    n, d = x.shape
    cbs = jnp.stack([codebook0, codebook1, codebook2, codebook3])
    m_, k, s = cbs.shape
    cbt = cbs.transpose(0, 2, 1)

    rt = rotateMatrix.reshape(d, m_, s).transpose(1, 0, 2) * (2.0 * _LOG2E)
    bn1 = min(1024, n)
    xrs, reg = pl.pallas_call(
        _rot_reg_kernel,
        grid=(n // bn1, m_),
        in_specs=[
            pl.BlockSpec((bn1, d), lambda i, m: (i, 0)),
            pl.BlockSpec((1, d, s), lambda i, m: (m, 0, 0)),
            pl.BlockSpec((d, d), lambda i, m: (0, 0)),
        ],
        out_specs=[
            pl.BlockSpec((1, bn1, s), lambda i, m: (m, i, 0)),
            pl.BlockSpec((1, 1), lambda i, m: (0, 0)),
        ],
        out_shape=[
            jax.ShapeDtypeStruct((m_, n, s), jnp.float32),
            jax.ShapeDtypeStruct((1, 1), jnp.float32),
        ],
    )(x, rt, rotateMatrix)
    bias = (-_LOG2E) * jnp.sum(cbs * cbs, axis=-1)[:, None, :]

    spad = 128
    cba = jnp.concatenate(
        [cbs.astype(jnp.bfloat16),
         jnp.ones((m_, k, 1), jnp.bfloat16),
         jnp.zeros((m_, k, spad - s - 1), jnp.bfloat16)], axis=2)
    bn = min(256, n)
    codes, soft = pl.pallas_call(
        _vq_kernel,
        grid=(m_, n // bn),
        in_specs=[
            pl.BlockSpec((1, bn, s), lambda m, i: (m, i, 0)),
            pl.BlockSpec((1, s, k), lambda m, i: (m, 0, 0)),
            pl.BlockSpec((1, 1, k), lambda m, i: (m, 0, 0)),
            pl.BlockSpec((1, k, spad), lambda m, i: (m, 0, 0)),
        ],
        out_specs=[
            pl.BlockSpec((1, 1, bn), lambda m, i: (m, 0, i)),
            pl.BlockSpec((1, bn, s), lambda m, i: (m, i, 0)),
        ],
        out_shape=[
            jax.ShapeDtypeStruct((m_, 1, n), jnp.int32),
            jax.ShapeDtypeStruct((m_, n, s), jnp.float32),
        ],
    )(xrs, cbt, bias, cba)

    codes_flat = codes.reshape(m_ * n)
    table = jnp.concatenate(
        [cbs.reshape(m_ * k, s), jnp.zeros((m_ * k, spad - s), jnp.float32)], axis=1)
    hard = _make_sc_gather(m_ * n, spad, n, k)(codes_flat, table)
    hard = hard.reshape(m_, n, spad)

    bn3 = min(2048, n)
    loss = pl.pallas_call(
        _loss_kernel,
        grid=(m_, n // bn3),
        in_specs=[
            pl.BlockSpec((1, bn3, s), lambda m, i: (m, i, 0)),
            pl.BlockSpec((1, bn3, s), lambda m, i: (m, i, 0)),
            pl.BlockSpec((1, bn3, spad), lambda m, i: (m, i, 0)),
            pl.BlockSpec((1, 1), lambda m, i: (0, 0)),
        ],
        out_specs=pl.BlockSpec((1, 1), lambda m, i: (0, 0)),
        out_shape=jax.ShapeDtypeStruct((1, 1), jnp.float32),
        scratch_shapes=[pltpu.SMEM((3,), jnp.float32)],
    )(xrs, soft, hard, reg)

    hard_codes = codes.reshape(m_, n).T
    return (hard_codes, loss[0, 0])
